# baseline (speedup 1.0000x reference)
import numpy as np
import jax
import jax.numpy as jnp
from jax import lax
from jax.experimental import pallas as pl
from jax.experimental.pallas import tpu as pltpu
from jax.experimental.pallas import tpu_sc as plsc

DIM = 64
WDIM = DIM // 2
T = 50
B = 4096
NC = 2
NS = 16
NW = NC * NS
RPW = B // NW
CB = 2
CHUNK = CB * T
NCHUNK = RPW // CB
LANES = 16
NG = DIM // 32
NBUF = 8
NGRP = NCHUNK // NBUF

_PERM = np.concatenate([
    np.concatenate([np.arange(g * 32, (g + 1) * 32, 2),
                    np.arange(g * 32 + 1, (g + 1) * 32, 2)])
    for g in range(NG)])
_INV_PERM = np.argsort(_PERM)

_GDN = lax.GatherDimensionNumbers(
    offset_dims=(), collapsed_slice_dims=(0,), start_index_map=(0,))


def _bcast_lane(vec, lane):
    idx = jnp.full((LANES, 1), lane, jnp.int32)
    return lax.gather(vec, idx, _GDN, (1,),
                      mode=lax.GatherScatterMode.PROMISE_IN_BOUNDS)


def _split_word(w):
    lo = lax.bitcast_convert_type(lax.shift_left(w, 16), jnp.float32)
    hi = lax.bitcast_convert_type(
        lax.bitwise_and(w, jnp.int32(-65536)), jnp.float32)
    return lo, hi


def _body(tbl, idx, pos, out, idx_v, pos_v, buf_v, out_v, emb0_v, sem):
    wid = lax.axis_index("s") * NC + lax.axis_index("c")
    pltpu.sync_copy(idx.at[wid], idx_v)
    pltpu.sync_copy(pos, pos_v)
    pltpu.sync_copy(tbl.at[pl.ds(0, 1)], emb0_v)

    def compute_chunk(c, buf):
        row0 = c * CB
        for g in range(NG):
            ae = [jnp.zeros((LANES,), jnp.float32) for _ in range(CB)]
            ao = [jnp.zeros((LANES,), jnp.float32) for _ in range(CB)]
            for t in range(T):
                pe = pos_v[pl.ds(t * DIM + g * 32, LANES)]
                po = pos_v[pl.ds(t * DIM + g * 32 + LANES, LANES)]
                for lb in range(CB):
                    w = buf[lb * T + t, pl.ds(g * LANES, LANES)]
                    vlo, vhi = _split_word(w)
                    ae[lb] = ae[lb] + vlo * pe
                    ao[lb] = ao[lb] + vhi * po
            for lb in range(CB):
                base = (row0 + lb) * DIM + g * 32
                out_v[pl.ds(base, LANES)] = ae[lb] * jnp.float32(1.0 / T)
                out_v[pl.ds(base + LANES, LANES)] = ao[lb] * jnp.float32(1.0 / T)

    for b in range(NBUF):
        pltpu.async_copy(tbl.at[idx_v.at[b]], buf_v.at[b], sem.at[b])

    def grp_body(grp, carry):
        c0 = grp * NBUF
        for b in range(NBUF):
            c = c0 + b
            pltpu.make_async_copy(
                tbl.at[idx_v.at[c]], buf_v.at[b], sem.at[b]).wait()
            compute_chunk(c, buf_v.at[b])
            nxt = c + NBUF

            @pl.when(nxt < NCHUNK)
            def _():
                pltpu.async_copy(
                    tbl.at[idx_v.at[nxt]], buf_v.at[b], sem.at[b])
        return carry

    lax.fori_loop(0, NGRP, grp_body, 0)

    def corr_body(r, carry):
        c = r // CB
        tb = (r % CB) * T
        ivs = [idx_v[c, pl.ds(tb + off, LANES)] for off in (0, 16, 32, 34)]
        pads = [iv == 0 for iv in ivs]
        anyp = jnp.any(pads[0] | pads[1] | pads[2] | pads[3])

        @pl.when(anyp)
        def _():
            g = [jnp.where(pv, jnp.float32(1), jnp.float32(0)) for pv in pads]
            corr = [jnp.zeros((LANES,), jnp.float32) for _ in range(2 * NG)]
            for t in range(T):
                gi, lane = (t // 16, t % 16) if t < 48 else (3, t - 34)
                mb = _bcast_lane(g[gi], lane)
                for h in range(2 * NG):
                    p = pos_v[pl.ds(t * DIM + h * LANES, LANES)]
                    corr[h] = corr[h] + p * mb
            for gg in range(NG):
                ew = emb0_v[0, pl.ds(gg * LANES, LANES)]
                elo, ehi = _split_word(ew)
                base = r * DIM + gg * 32
                o0 = out_v[pl.ds(base, LANES)]
                out_v[pl.ds(base, LANES)] = (
                    o0 - elo * corr[2 * gg] * jnp.float32(1.0 / T))
                o1 = out_v[pl.ds(base + LANES, LANES)]
                out_v[pl.ds(base + LANES, LANES)] = (
                    o1 - ehi * corr[2 * gg + 1] * jnp.float32(1.0 / T))
        return carry

    lax.fori_loop(0, RPW, corr_body, 0)
    pltpu.sync_copy(out_v, out.at[wid])


def kernel(input, emb_table, pos_table):
    idx = input.reshape(NW, NCHUNK, CHUNK).astype(jnp.int32)
    pos = pos_table[:, _PERM].reshape(T * DIM)
    tblw = lax.bitcast_convert_type(
        emb_table.astype(jnp.bfloat16).reshape(-1, WDIM, 2),
        jnp.int32)
    mesh = plsc.VectorSubcoreMesh(core_axis_name="c", subcore_axis_name="s",
                                  num_cores=NC, num_subcores=NS)
    out = pl.kernel(
        _body,
        out_type=jax.ShapeDtypeStruct((NW, RPW * DIM), jnp.float32),
        mesh=mesh,
        scratch_types=[
            pltpu.VMEM((NCHUNK, CHUNK), jnp.int32),
            pltpu.VMEM((T * DIM,), jnp.float32),
            pltpu.VMEM((NBUF, CHUNK, WDIM), jnp.int32),
            pltpu.VMEM((RPW * DIM,), jnp.float32),
            pltpu.VMEM((1, WDIM), jnp.int32),
            pltpu.SemaphoreType.DMA((NBUF,)),
        ],
        compiler_params=pltpu.CompilerParams(use_tc_tiling_on_sc=False,
                                             needs_layout_passes=False),
    )(tblw, idx, pos)
    return out.reshape(B, DIM)[:, _INV_PERM]


# speedup vs baseline: 2.5621x; 2.5621x over previous
import jax
import jax.numpy as jnp
from jax import lax
from jax.experimental import pallas as pl
from jax.experimental.pallas import tpu as pltpu
from jax.experimental.pallas import tpu_sc as plsc

DIM = 64
T = 50
B = 4096
NC = 2
NS = 16
NW = NC * NS
RPW = B // NW
CB = 2
CHUNK = CB * T
NCHUNK = RPW // CB
LANES = 16
NDC = DIM // LANES
NBUF = 4
NGRP = NCHUNK // NBUF

_GDN = lax.GatherDimensionNumbers(
    offset_dims=(), collapsed_slice_dims=(0,), start_index_map=(0,))


def _bcast_lane(vec, lane):
    idx = jnp.full((LANES, 1), lane, jnp.int32)
    return lax.gather(vec, idx, _GDN, (1,),
                      mode=lax.GatherScatterMode.PROMISE_IN_BOUNDS)


def _body(tbl, idx, pos, out, idx_v, pos_v, buf_v, out_v, emb0_v, sem):
    wid = lax.axis_index("s") * NC + lax.axis_index("c")
    pltpu.sync_copy(idx.at[wid], idx_v)
    pltpu.sync_copy(pos, pos_v)
    pltpu.sync_copy(tbl.at[pl.ds(0, 1)], emb0_v)

    def compute_chunk(c, buf):
        row0 = c * CB
        for dc in range(NDC):
            a = [jnp.zeros((LANES,), jnp.float32) for _ in range(CB)]
            for t in range(T):
                p = pos_v[pl.ds(t * DIM + dc * LANES, LANES)]
                for lb in range(CB):
                    v = buf[lb * T + t, pl.ds(dc * LANES, LANES)]
                    a[lb] = a[lb] + v * p
            for lb in range(CB):
                out_v[pl.ds((row0 + lb) * DIM + dc * LANES, LANES)] = (
                    a[lb] * jnp.float32(1.0 / T))

    for b in range(NBUF):
        pltpu.async_copy(tbl.at[idx_v.at[b]], buf_v.at[b], sem.at[b])

    def grp_body(grp, carry):
        c0 = grp * NBUF
        for b in range(NBUF):
            c = c0 + b
            pltpu.make_async_copy(
                tbl.at[idx_v.at[c]], buf_v.at[b], sem.at[b]).wait()
            compute_chunk(c, buf_v.at[b])
            nxt = c + NBUF

            @pl.when(nxt < NCHUNK)
            def _():
                pltpu.async_copy(
                    tbl.at[idx_v.at[nxt]], buf_v.at[b], sem.at[b])
        return carry

    lax.fori_loop(0, NGRP, grp_body, 0)

    def corr_body(r, carry):
        c = r // CB
        tb = (r % CB) * T
        ivs = [idx_v[c, pl.ds(tb + off, LANES)] for off in (0, 16, 32, 34)]
        pads = [iv == 0 for iv in ivs]
        anyp = jnp.any(pads[0] | pads[1] | pads[2] | pads[3])

        @pl.when(anyp)
        def _():
            g = [jnp.where(pv, jnp.float32(1), jnp.float32(0)) for pv in pads]
            corr = [jnp.zeros((LANES,), jnp.float32) for _ in range(NDC)]
            for t in range(T):
                gi, lane = (t // 16, t % 16) if t < 48 else (3, t - 34)
                mb = _bcast_lane(g[gi], lane)
                for dc in range(NDC):
                    p = pos_v[pl.ds(t * DIM + dc * LANES, LANES)]
                    corr[dc] = corr[dc] + p * mb
            for dc in range(NDC):
                e0 = emb0_v[0, pl.ds(dc * LANES, LANES)]
                o = out_v[pl.ds(r * DIM + dc * LANES, LANES)]
                out_v[pl.ds(r * DIM + dc * LANES, LANES)] = (
                    o - e0 * corr[dc] * jnp.float32(1.0 / T))
        return carry

    lax.fori_loop(0, RPW, corr_body, 0)
    pltpu.sync_copy(out_v, out.at[wid])


def kernel(input, emb_table, pos_table):
    idx = input.reshape(NW, NCHUNK, CHUNK).astype(jnp.int32)
    pos = pos_table.reshape(T * DIM)
    mesh = plsc.VectorSubcoreMesh(core_axis_name="c", subcore_axis_name="s",
                                  num_cores=NC, num_subcores=NS)
    out = pl.kernel(
        _body,
        out_type=jax.ShapeDtypeStruct((NW, RPW * DIM), jnp.float32),
        mesh=mesh,
        scratch_types=[
            pltpu.VMEM((NCHUNK, CHUNK), jnp.int32),
            pltpu.VMEM((T * DIM,), jnp.float32),
            pltpu.VMEM((NBUF, CHUNK, DIM), jnp.float32),
            pltpu.VMEM((RPW * DIM,), jnp.float32),
            pltpu.VMEM((1, DIM), jnp.float32),
            pltpu.SemaphoreType.DMA((NBUF,)),
        ],
        compiler_params=pltpu.CompilerParams(use_tc_tiling_on_sc=False,
                                             needs_layout_passes=False),
    )(emb_table, idx, pos)
    return out.reshape(B, DIM)


# speedup vs baseline: 2.6645x; 1.0400x over previous
import jax
import jax.numpy as jnp
from jax import lax
from jax.experimental import pallas as pl
from jax.experimental.pallas import tpu as pltpu
from jax.experimental.pallas import tpu_sc as plsc

DIM = 64
T = 50
B = 4096
NC = 2
NS = 16
NW = NC * NS
RPW = B // NW
CB = 2
CHUNK = CB * T
NCHUNK = RPW // CB
LANES = 16
NDC = DIM // LANES
NBUF = 2
NGRP = NCHUNK // NBUF

_GDN = lax.GatherDimensionNumbers(
    offset_dims=(), collapsed_slice_dims=(0,), start_index_map=(0,))


def _bcast_lane(vec, lane):
    idx = jnp.full((LANES, 1), lane, jnp.int32)
    return lax.gather(vec, idx, _GDN, (1,),
                      mode=lax.GatherScatterMode.PROMISE_IN_BOUNDS)


def _body(tbl, idx, pos, out, idx_v, pos_v, buf_v, out_v, emb0_v, sem):
    wid = lax.axis_index("s") * NC + lax.axis_index("c")
    pltpu.sync_copy(idx.at[wid], idx_v)
    pltpu.sync_copy(pos, pos_v)
    pltpu.sync_copy(tbl.at[pl.ds(0, 1)], emb0_v)

    def compute_chunk(c, buf):
        row0 = c * CB
        for dc in range(NDC):
            a = [jnp.zeros((LANES,), jnp.float32) for _ in range(CB)]
            for t in range(T):
                p = pos_v[pl.ds(t * DIM + dc * LANES, LANES)]
                for lb in range(CB):
                    v = buf[lb * T + t, pl.ds(dc * LANES, LANES)]
                    a[lb] = a[lb] + v * p
            for lb in range(CB):
                out_v[pl.ds((row0 + lb) * DIM + dc * LANES, LANES)] = (
                    a[lb] * jnp.float32(1.0 / T))

    for b in range(NBUF):
        pltpu.async_copy(tbl.at[idx_v.at[b]], buf_v.at[b], sem.at[b])

    def grp_body(grp, carry):
        c0 = grp * NBUF
        for b in range(NBUF):
            c = c0 + b
            pltpu.make_async_copy(
                tbl.at[idx_v.at[c]], buf_v.at[b], sem.at[b]).wait()
            compute_chunk(c, buf_v.at[b])
            nxt = c + NBUF

            @pl.when(nxt < NCHUNK)
            def _():
                pltpu.async_copy(
                    tbl.at[idx_v.at[nxt]], buf_v.at[b], sem.at[b])
        return carry

    lax.fori_loop(0, NGRP, grp_body, 0)

    def corr_body(r, carry):
        c = r // CB
        tb = (r % CB) * T
        ivs = [idx_v[c, pl.ds(tb + off, LANES)] for off in (0, 16, 32, 34)]
        pads = [iv == 0 for iv in ivs]
        anyp = jnp.any(pads[0] | pads[1] | pads[2] | pads[3])

        @pl.when(anyp)
        def _():
            g = [jnp.where(pv, jnp.float32(1), jnp.float32(0)) for pv in pads]
            corr = [jnp.zeros((LANES,), jnp.float32) for _ in range(NDC)]
            for t in range(T):
                gi, lane = (t // 16, t % 16) if t < 48 else (3, t - 34)
                mb = _bcast_lane(g[gi], lane)
                for dc in range(NDC):
                    p = pos_v[pl.ds(t * DIM + dc * LANES, LANES)]
                    corr[dc] = corr[dc] + p * mb
            for dc in range(NDC):
                e0 = emb0_v[0, pl.ds(dc * LANES, LANES)]
                o = out_v[pl.ds(r * DIM + dc * LANES, LANES)]
                out_v[pl.ds(r * DIM + dc * LANES, LANES)] = (
                    o - e0 * corr[dc] * jnp.float32(1.0 / T))
        return carry

    lax.fori_loop(0, RPW, corr_body, 0)
    pltpu.sync_copy(out_v, out.at[wid])


def kernel(input, emb_table, pos_table):
    idx = input.reshape(NW, NCHUNK, CHUNK).astype(jnp.int32)
    pos = pos_table.reshape(T * DIM)
    mesh = plsc.VectorSubcoreMesh(core_axis_name="c", subcore_axis_name="s",
                                  num_cores=NC, num_subcores=NS)
    out = pl.kernel(
        _body,
        out_type=jax.ShapeDtypeStruct((NW, RPW * DIM), jnp.float32),
        mesh=mesh,
        scratch_types=[
            pltpu.VMEM((NCHUNK, CHUNK), jnp.int32),
            pltpu.VMEM((T * DIM,), jnp.float32),
            pltpu.VMEM((NBUF, CHUNK, DIM), jnp.float32),
            pltpu.VMEM((RPW * DIM,), jnp.float32),
            pltpu.VMEM((1, DIM), jnp.float32),
            pltpu.SemaphoreType.DMA((NBUF,)),
        ],
        compiler_params=pltpu.CompilerParams(use_tc_tiling_on_sc=False,
                                             needs_layout_passes=False),
    )(emb_table, idx, pos)
    return out.reshape(B, DIM)


# speedup vs baseline: 2.7363x; 1.0269x over previous
import jax
import jax.numpy as jnp
from jax import lax
from jax.experimental import pallas as pl
from jax.experimental.pallas import tpu as pltpu
from jax.experimental.pallas import tpu_sc as plsc

DIM = 64
T = 50
B = 4096
NC = 2
NS = 16
NW = NC * NS
RPW = B // NW
CB = 2
CHUNK = CB * T
NCHUNK = RPW // CB
LANES = 16
NDC = DIM // LANES
NBUF = 4
NGRP = NCHUNK // NBUF

_GDN = lax.GatherDimensionNumbers(
    offset_dims=(), collapsed_slice_dims=(0,), start_index_map=(0,))


def _bcast_lane(vec, lane):
    idx = jnp.full((LANES, 1), lane, jnp.int32)
    return lax.gather(vec, idx, _GDN, (1,),
                      mode=lax.GatherScatterMode.PROMISE_IN_BOUNDS)


def _body(tbl, idx, pos, out, idx_v, pos_v, buf_v, out_v, emb0_v, m_v, sem):
    wid = lax.axis_index("s") * NC + lax.axis_index("c")
    pltpu.sync_copy(idx.at[wid], idx_v)
    pltpu.sync_copy(pos, pos_v)
    pltpu.sync_copy(tbl.at[pl.ds(0, 1)], emb0_v)

    def compute_chunk(c, buf):
        row0 = c * CB

        def t_body(t, accs):
            accs = list(accs)
            for dc in range(NDC):
                p = pos_v[pl.ds(t * DIM + dc * LANES, LANES)]
                for lb in range(CB):
                    v = buf[lb * T + t, pl.ds(dc * LANES, LANES)]
                    k = dc * CB + lb
                    accs[k] = accs[k] + v * p
            return tuple(accs)

        init = tuple(jnp.zeros((LANES,), jnp.float32)
                     for _ in range(NDC * CB))
        accs = lax.fori_loop(0, T, t_body, init)
        for dc in range(NDC):
            for lb in range(CB):
                out_v[pl.ds((row0 + lb) * DIM + dc * LANES, LANES)] = (
                    accs[dc * CB + lb] * jnp.float32(1.0 / T))

    for b in range(NBUF):
        pltpu.async_copy(tbl.at[idx_v.at[b]], buf_v.at[b], sem.at[b])

    def grp_body(grp, carry):
        c0 = grp * NBUF
        for b in range(NBUF):
            c = c0 + b
            pltpu.make_async_copy(
                tbl.at[idx_v.at[c]], buf_v.at[b], sem.at[b]).wait()
            compute_chunk(c, buf_v.at[b])
            nxt = c + NBUF

            @pl.when(nxt < NCHUNK)
            def _():
                pltpu.async_copy(
                    tbl.at[idx_v.at[nxt]], buf_v.at[b], sem.at[b])
        return carry

    lax.fori_loop(0, NGRP, grp_body, 0)

    def corr_body(r, carry):
        c = r // CB
        tb = (r % CB) * T
        ivs = [idx_v[c, pl.ds(tb + off, LANES)] for off in (0, 16, 32, 34)]
        pads = [iv == 0 for iv in ivs]
        anyp = jnp.any(pads[0] | pads[1] | pads[2] | pads[3])

        @pl.when(anyp)
        def _():
            g = [jnp.where(pv, jnp.float32(1), jnp.float32(0)) for pv in pads]
            for k in range(4):
                m_v[pl.ds(k * LANES, LANES)] = g[k]

            def ct_body(t, corr):
                corr = list(corr)
                tpos = jnp.where(t >= 48, t + 14, t)
                mb = _bcast_lane(m_v[pl.ds(tpos, LANES)], 0)
                for dc in range(NDC):
                    p = pos_v[pl.ds(t * DIM + dc * LANES, LANES)]
                    corr[dc] = corr[dc] + p * mb
                return tuple(corr)

            corr = lax.fori_loop(
                0, T, ct_body,
                tuple(jnp.zeros((LANES,), jnp.float32) for _ in range(NDC)))
            for dc in range(NDC):
                e0 = emb0_v[0, pl.ds(dc * LANES, LANES)]
                o = out_v[pl.ds(r * DIM + dc * LANES, LANES)]
                out_v[pl.ds(r * DIM + dc * LANES, LANES)] = (
                    o - e0 * corr[dc] * jnp.float32(1.0 / T))
        return carry

    lax.fori_loop(0, RPW, corr_body, 0)
    pltpu.sync_copy(out_v, out.at[wid])


def kernel(input, emb_table, pos_table):
    idx = input.reshape(NW, NCHUNK, CHUNK).astype(jnp.int32)
    pos = pos_table.reshape(T * DIM)
    mesh = plsc.VectorSubcoreMesh(core_axis_name="c", subcore_axis_name="s",
                                  num_cores=NC, num_subcores=NS)
    out = pl.kernel(
        _body,
        out_type=jax.ShapeDtypeStruct((NW, RPW * DIM), jnp.float32),
        mesh=mesh,
        scratch_types=[
            pltpu.VMEM((NCHUNK, CHUNK), jnp.int32),
            pltpu.VMEM((T * DIM,), jnp.float32),
            pltpu.VMEM((NBUF, CHUNK, DIM), jnp.float32),
            pltpu.VMEM((RPW * DIM,), jnp.float32),
            pltpu.VMEM((1, DIM), jnp.float32),
            pltpu.VMEM((80,), jnp.float32),
            pltpu.SemaphoreType.DMA((NBUF,)),
        ],
        compiler_params=pltpu.CompilerParams(use_tc_tiling_on_sc=False,
                                             needs_layout_passes=False),
    )(emb_table, idx, pos)
    return out.reshape(B, DIM)
